# Initial kernel scaffold; baseline (speedup 1.0000x reference)
import functools

import jax
import jax.numpy as jnp
from jax import lax
from jax.experimental import pallas as pl
from jax.experimental.pallas import tpu as pltpu
from jax.experimental.pallas import tpu_sc as plsc

N = 10000
E = 320000
IN_DIM = 128
HEADS = 2
OUT = 64
HC = HEADS * OUT

ET = E + N
TILES = 32
CHUNK = 128
CPT = -(-ET // (TILES * CHUNK))
EP = TILES * CPT * CHUNK
ROWS_PER_SUB = N // 16


def _prologue_body(x_ref, w_ref, as_ref, ad_ref, h_ref, asrc_ref, adst_ref, amax_ref):
    xb = x_ref[...]
    xf = jnp.where(xb > 0, xb, jnp.expm1(xb))
    h = jnp.dot(xf, w_ref[...], preferred_element_type=jnp.float32)
    h_ref[...] = h
    ts = h * as_ref[...]
    td = h * ad_ref[...]
    a0 = jnp.sum(ts[:, :OUT], axis=1, keepdims=True)
    a1 = jnp.sum(ts[:, OUT:], axis=1, keepdims=True)
    d0 = jnp.sum(td[:, :OUT], axis=1, keepdims=True)
    d1 = jnp.sum(td[:, OUT:], axis=1, keepdims=True)
    asrc_ref[...] = jnp.concatenate([a0, a1], axis=1)
    adst_ref[...] = jnp.concatenate([d0, d1], axis=1)

    @pl.when(pl.program_id(0) == 0)
    def _():
        amax_ref[...] = jnp.full((2, 16), -1e30, jnp.float32)

    upd = jnp.concatenate(
        [jnp.full((1, 16), jnp.max(a0), jnp.float32),
         jnp.full((1, 16), jnp.max(a1), jnp.float32)], axis=0)
    amax_ref[...] = jnp.maximum(amax_ref[...], upd)


def _prologue(x, W, atts, attd):
    RB = 1000
    return pl.pallas_call(
        _prologue_body,
        grid=(N // RB,),
        in_specs=[pl.BlockSpec((RB, IN_DIM), lambda i: (i, 0)),
                  pl.BlockSpec((IN_DIM, HC), lambda i: (0, 0)),
                  pl.BlockSpec((1, HC), lambda i: (0, 0)),
                  pl.BlockSpec((1, HC), lambda i: (0, 0))],
        out_specs=[pl.BlockSpec((RB, HC), lambda i: (i, 0)),
                   pl.BlockSpec((RB, 2), lambda i: (i, 0)),
                   pl.BlockSpec((RB, 2), lambda i: (i, 0)),
                   pl.BlockSpec((2, 16), lambda i: (0, 0))],
        out_shape=[jax.ShapeDtypeStruct((N, HC), jnp.float32),
                   jax.ShapeDtypeStruct((N, 2), jnp.float32),
                   jax.ShapeDtypeStruct((N, 2), jnp.float32),
                   jax.ShapeDtypeStruct((2, 16), jnp.float32)],
    )(x, W, atts, attd)


def _edge_softmax(asrc, adst, amax32, srcp, dstp, z2):
    mesh = plsc.VectorSubcoreMesh(core_axis_name="c", subcore_axis_name="s")

    @functools.partial(
        pl.kernel,
        out_type=[jax.ShapeDtypeStruct((EP,), jnp.float32),
                  jax.ShapeDtypeStruct((EP,), jnp.float32),
                  jax.ShapeDtypeStruct((2 * N, 2), jnp.float32)],
        mesh=mesh,
        scratch_types=[pltpu.VMEM((N, 2), jnp.float32),
                       pltpu.VMEM((N, 2), jnp.float32),
                       pltpu.VMEM((32,), jnp.float32),
                       pltpu.VMEM((CHUNK,), jnp.int32),
                       pltpu.VMEM((CHUNK,), jnp.int32),
                       pltpu.VMEM((CHUNK,), jnp.float32),
                       pltpu.VMEM((CHUNK,), jnp.float32),
                       pltpu.VMEM((CHUNK, 2), jnp.float32),
                       pltpu.VMEM_SHARED((N, 2), jnp.float32)],
    )
    def k(asrc_h, adst_h, amax_h, src_h, dst_h, z2_h, ex0_h, ex1_h, den_h,
          asv, adv, amv, srcb, dstb, ex0b, ex1b, exI, den_sp):
        ci = lax.axis_index("c")
        si = lax.axis_index("s")
        t = ci * 16 + si
        pltpu.sync_copy(asrc_h, asv)
        pltpu.sync_copy(adst_h, adv)
        pltpu.sync_copy(amax_h, amv)

        @pl.when(si < 10)
        def _():
            r0 = si * 1000
            pltpu.sync_copy(z2_h.at[pl.ds(r0, 1000)], den_sp.at[pl.ds(r0, 1000)])

        plsc.subcore_barrier()

        am0 = amv[pl.ds(0, 16)]
        am1 = amv[pl.ds(16, 16)]
        z16 = jnp.zeros((16,), jnp.int32)
        o16 = jnp.full((16,), 1, jnp.int32)
        iota = lax.iota(jnp.int32, 16)

        def chunk_body(c, carry):
            base = t * (CPT * CHUNK) + c * CHUNK
            pltpu.sync_copy(src_h.at[pl.ds(base, CHUNK)], srcb)
            pltpu.sync_copy(dst_h.at[pl.ds(base, CHUNK)], dstb)
            for g in range(CHUNK // 16):
                b16 = g * 16
                s16 = srcb[pl.ds(b16, 16)]
                d16 = dstb[pl.ds(b16, 16)]
                as0 = plsc.load_gather(asv, [s16, z16])
                as1 = plsc.load_gather(asv, [s16, o16])
                ad0 = plsc.load_gather(adv, [d16, z16])
                ad1 = plsc.load_gather(adv, [d16, o16])
                al0 = as0 + ad0
                al1 = as1 + ad1
                al0 = jnp.maximum(al0, 0.2 * al0)
                al1 = jnp.maximum(al1, 0.2 * al1)
                b0 = am0 + ad0
                b1 = am1 + ad1
                b0 = jnp.maximum(b0, 0.2 * b0)
                b1 = jnp.maximum(b1, 0.2 * b1)
                e0 = jnp.exp(al0 - b0)
                e1 = jnp.exp(al1 - b1)
                eid = base + b16 + iota
                msk = eid < ET
                e0 = jnp.where(msk, e0, 0.0)
                e1 = jnp.where(msk, e1, 0.0)
                ex0b[pl.ds(b16, 16)] = e0
                ex1b[pl.ds(b16, 16)] = e1
                lane = b16 + iota
                plsc.store_scatter(exI, [lane, z16], e0)
                plsc.store_scatter(exI, [lane, o16], e1)
            pltpu.sync_copy(ex0b, ex0_h.at[pl.ds(base, CHUNK)])
            pltpu.sync_copy(ex1b, ex1_h.at[pl.ds(base, CHUNK)])
            pltpu.sync_copy(exI, den_sp.at[dstb], add=True)
            return carry

        lax.fori_loop(0, CPT, chunk_body, 0)
        plsc.subcore_barrier()

        @pl.when(si < 10)
        def _():
            r0 = si * 1000
            pltpu.sync_copy(den_sp.at[pl.ds(r0, 1000)],
                            den_h.at[pl.ds(ci * N + r0, 1000)])

    return k(asrc, adst, amax32, srcp, dstp, z2)


def _message_pass(h, srcp, dstp, ex0, ex1, zbig):
    mesh = plsc.VectorSubcoreMesh(core_axis_name="c", subcore_axis_name="s")

    @functools.partial(
        pl.kernel,
        out_type=jax.ShapeDtypeStruct((2 * N, HC), jnp.float32),
        mesh=mesh,
        scratch_types=[pltpu.VMEM((CHUNK,), jnp.int32),
                       pltpu.VMEM((CHUNK,), jnp.int32),
                       pltpu.VMEM((CHUNK,), jnp.float32),
                       pltpu.VMEM((CHUNK,), jnp.float32),
                       pltpu.VMEM((CHUNK, HC), jnp.float32),
                       pltpu.VMEM_SHARED((N, HC), jnp.float32),
                       pltpu.SemaphoreType.DMA],
    )
    def k(h_h, src_h, dst_h, ex0_h, ex1_h, z_h, out_h,
          srcb, dstb, ex0b, ex1b, rows, acc_sp, sem):
        ci = lax.axis_index("c")
        si = lax.axis_index("s")
        t = ci * 16 + si
        r0 = si * ROWS_PER_SUB
        pltpu.sync_copy(z_h.at[pl.ds(r0, ROWS_PER_SUB)],
                        acc_sp.at[pl.ds(r0, ROWS_PER_SUB)])
        plsc.subcore_barrier()

        def chunk_body(c, carry):
            base = t * (CPT * CHUNK) + c * CHUNK
            pltpu.sync_copy(src_h.at[pl.ds(base, CHUNK)], srcb)
            pltpu.sync_copy(dst_h.at[pl.ds(base, CHUNK)], dstb)
            pltpu.sync_copy(ex0_h.at[pl.ds(base, CHUNK)], ex0b)
            pltpu.sync_copy(ex1_h.at[pl.ds(base, CHUNK)], ex1b)
            pltpu.async_copy(h_h.at[srcb], rows, sem).wait()

            def e_body(e, c2):
                s0 = ex0b[e]
                s1 = ex1b[e]
                for j in range(HC // 16):
                    sc = s0 if j < (HC // 32) else s1
                    rows[e, pl.ds(j * 16, 16)] = rows[e, pl.ds(j * 16, 16)] * sc
                return c2

            lax.fori_loop(0, CHUNK, e_body, 0)
            pltpu.sync_copy(rows, acc_sp.at[dstb], add=True)
            return carry

        lax.fori_loop(0, CPT, chunk_body, 0)
        plsc.subcore_barrier()
        pltpu.sync_copy(acc_sp.at[pl.ds(r0, ROWS_PER_SUB)],
                        out_h.at[pl.ds(ci * N + r0, ROWS_PER_SUB)])

    return k(h, srcp, dstp, ex0, ex1, zbig)


def _finalize_body(p0_ref, p1_ref, d0_ref, d1_ref, b_ref, o_ref):
    den = d0_ref[...] + d1_ref[...]
    p = p0_ref[...] + p1_ref[...]
    cols = lax.broadcasted_iota(jnp.int32, p.shape, 1)
    den0 = jnp.broadcast_to(den[:, 0:1], p.shape)
    den1 = jnp.broadcast_to(den[:, 1:2], p.shape)
    db = jnp.where(cols < OUT, den0, den1)
    o_ref[...] = p / (db + 1e-16) + b_ref[...]


def _finalize(P, denp, bias2):
    RB = 1000
    G = N // RB
    return pl.pallas_call(
        _finalize_body,
        grid=(G,),
        in_specs=[pl.BlockSpec((RB, HC), lambda i: (i, 0)),
                  pl.BlockSpec((RB, HC), lambda i: (i + G, 0)),
                  pl.BlockSpec((RB, 2), lambda i: (i, 0)),
                  pl.BlockSpec((RB, 2), lambda i: (i + G, 0)),
                  pl.BlockSpec((1, HC), lambda i: (0, 0))],
        out_specs=pl.BlockSpec((RB, HC), lambda i: (i, 0)),
        out_shape=jax.ShapeDtypeStruct((N, HC), jnp.float32),
    )(P, P, denp, denp, bias2)


def kernel(x, edge_index, W, att_src, att_dst, bias):
    h, asrc, adst, amax = _prologue(x, W, att_src.reshape(1, HC),
                                    att_dst.reshape(1, HC))
    loops = jnp.arange(N, dtype=jnp.int32)
    pad = jnp.zeros((EP - ET,), jnp.int32)
    srcp = jnp.concatenate([edge_index[0], loops, pad])
    dstp = jnp.concatenate([edge_index[1], loops, pad])
    ex0, ex1, denp = _edge_softmax(asrc, adst, amax.reshape(32), srcp, dstp,
                                   jnp.zeros((N, 2), jnp.float32))
    P = _message_pass(h, srcp, dstp, ex0, ex1, jnp.zeros((N, HC), jnp.float32))
    return _finalize(P, denp, bias.reshape(1, HC))


# speedup vs baseline: 61.3974x; 61.3974x over previous
import functools

import jax
import jax.numpy as jnp
from jax import lax
from jax.experimental import pallas as pl
from jax.experimental.pallas import tpu as pltpu
from jax.experimental.pallas import tpu_sc as plsc

N = 10000
E = 320000
IN_DIM = 128
HEADS = 2
OUT = 64
HC = HEADS * OUT

ET = E + N
TILES = 32
CHUNK = 128
CPT = 82
EP = TILES * CPT * CHUNK
NPAIR = CPT // 2
ROWS_PER_SUB = 624
DEN_PAD = 20480
DEN_PER_SUB = DEN_PAD // 16


def _prologue_body(x_ref, w_ref, as_ref, ad_ref, h_ref, asrc_ref, adst_ref, amax_ref):
    xb = x_ref[...]
    xf = jnp.where(xb > 0, xb, jnp.exp(xb) - 1.0)
    h = jnp.dot(xf, w_ref[...], preferred_element_type=jnp.float32)
    h_ref[...] = h
    ts = h * as_ref[...]
    td = h * ad_ref[...]
    a0 = jnp.sum(ts[:, :OUT], axis=1, keepdims=True)
    a1 = jnp.sum(ts[:, OUT:], axis=1, keepdims=True)
    d0 = jnp.sum(td[:, :OUT], axis=1, keepdims=True)
    d1 = jnp.sum(td[:, OUT:], axis=1, keepdims=True)
    asrc_ref[...] = jnp.concatenate([a0, a1], axis=1)
    adst_ref[...] = jnp.concatenate([d0, d1], axis=1)

    @pl.when(pl.program_id(0) == 0)
    def _():
        amax_ref[...] = jnp.full((2, 16), -1e30, jnp.float32)

    upd = jnp.concatenate(
        [jnp.full((1, 16), jnp.max(a0), jnp.float32),
         jnp.full((1, 16), jnp.max(a1), jnp.float32)], axis=0)
    amax_ref[...] = jnp.maximum(amax_ref[...], upd)


def _prologue(x, W, atts, attd):
    RB = 1000
    return pl.pallas_call(
        _prologue_body,
        grid=(N // RB,),
        in_specs=[pl.BlockSpec((RB, IN_DIM), lambda i: (i, 0)),
                  pl.BlockSpec((IN_DIM, HC), lambda i: (0, 0)),
                  pl.BlockSpec((1, HC), lambda i: (0, 0)),
                  pl.BlockSpec((1, HC), lambda i: (0, 0))],
        out_specs=[pl.BlockSpec((RB, HC), lambda i: (i, 0)),
                   pl.BlockSpec((RB, 2), lambda i: (i, 0)),
                   pl.BlockSpec((RB, 2), lambda i: (i, 0)),
                   pl.BlockSpec((2, 16), lambda i: (0, 0))],
        out_shape=[jax.ShapeDtypeStruct((N, HC), jnp.float32),
                   jax.ShapeDtypeStruct((N, 2), jnp.float32),
                   jax.ShapeDtypeStruct((N, 2), jnp.float32),
                   jax.ShapeDtypeStruct((2, 16), jnp.float32)],
    )(x, W, atts, attd)


def _edge_softmax(asrc2, adst2, amax32, srcp, dstp):
    mesh = plsc.VectorSubcoreMesh(core_axis_name="c", subcore_axis_name="s")

    @functools.partial(
        pl.kernel,
        out_type=[jax.ShapeDtypeStruct((EP,), jnp.float32),
                  jax.ShapeDtypeStruct((EP,), jnp.float32),
                  jax.ShapeDtypeStruct((2 * DEN_PAD,), jnp.float32)],
        mesh=mesh,
        compiler_params=pltpu.CompilerParams(needs_layout_passes=False),
        scratch_types=[pltpu.VMEM((2 * N,), jnp.float32),
                       pltpu.VMEM((2 * N,), jnp.float32),
                       pltpu.VMEM((32,), jnp.float32),
                       pltpu.VMEM((CHUNK,), jnp.int32),
                       pltpu.VMEM((CHUNK,), jnp.int32),
                       pltpu.VMEM((CHUNK,), jnp.float32),
                       pltpu.VMEM((CHUNK,), jnp.float32),
                       pltpu.VMEM((DEN_PAD,), jnp.float32),
                       pltpu.VMEM((DEN_PER_SUB,), jnp.float32),
                       pltpu.VMEM((DEN_PER_SUB,), jnp.float32),
                       pltpu.VMEM_SHARED((16 * DEN_PAD,), jnp.float32)],
    )
    def k(asrc_h, adst_h, amax_h, src_h, dst_h, ex0_h, ex1_h, den_h,
          asv, adv, amv, srcb, dstb, ex0b, ex1b, denv, accv, tmpv, den_sp):
        ci = lax.axis_index("c")
        si = lax.axis_index("s")
        t = ci * 16 + si
        pltpu.sync_copy(asrc_h, asv)
        pltpu.sync_copy(adst_h, adv)
        pltpu.sync_copy(amax_h, amv)

        z16f = jnp.zeros((16,), jnp.float32)

        def z_body(v, carry):
            denv[pl.ds(v * 16, 16)] = z16f
            return carry

        lax.fori_loop(0, DEN_PAD // 16, z_body, 0)

        am0 = amv[pl.ds(0, 16)]
        am1 = amv[pl.ds(16, 16)]
        iota = lax.iota(jnp.int32, 16)

        def chunk_body(c, carry):
            base = (t * CPT + c) * CHUNK
            pltpu.sync_copy(src_h.at[pl.ds(base, CHUNK)], srcb)
            pltpu.sync_copy(dst_h.at[pl.ds(base, CHUNK)], dstb)
            for g in range(CHUNK // 16):
                b16 = g * 16
                s16 = srcb[pl.ds(b16, 16)]
                d16 = dstb[pl.ds(b16, 16)]
                s2 = s16 * 2
                d2 = d16 * 2
                as0 = plsc.load_gather(asv, [s2])
                as1 = plsc.load_gather(asv, [s2 + 1])
                ad0 = plsc.load_gather(adv, [d2])
                ad1 = plsc.load_gather(adv, [d2 + 1])
                al0 = as0 + ad0
                al1 = as1 + ad1
                al0 = jnp.maximum(al0, 0.2 * al0)
                al1 = jnp.maximum(al1, 0.2 * al1)
                b0 = am0 + ad0
                b1 = am1 + ad1
                b0 = jnp.maximum(b0, 0.2 * b0)
                b1 = jnp.maximum(b1, 0.2 * b1)
                e0 = jnp.exp(al0 - b0)
                e1 = jnp.exp(al1 - b1)
                eid = base + b16 + iota
                msk = eid < ET
                e0 = jnp.where(msk, e0, 0.0)
                e1 = jnp.where(msk, e1, 0.0)
                ex0b[pl.ds(b16, 16)] = e0
                ex1b[pl.ds(b16, 16)] = e1
                plsc.addupdate_scatter(denv, [d2], e0)
                plsc.addupdate_scatter(denv, [d2 + 1], e1)
            pltpu.sync_copy(ex0b, ex0_h.at[pl.ds(base, CHUNK)])
            pltpu.sync_copy(ex1b, ex1_h.at[pl.ds(base, CHUNK)])
            return carry

        lax.fori_loop(0, CPT, chunk_body, 0)

        pltpu.sync_copy(denv, den_sp.at[pl.ds(si * DEN_PAD, DEN_PAD)])
        plsc.subcore_barrier()
        col0 = si * DEN_PER_SUB

        def za_body(v, carry):
            accv[pl.ds(v * 16, 16)] = z16f
            return carry

        lax.fori_loop(0, DEN_PER_SUB // 16, za_body, 0)

        def red_body(r, carry):
            pltpu.sync_copy(den_sp.at[pl.ds(r * DEN_PAD + col0, DEN_PER_SUB)], tmpv)

            def add_body(v, c2):
                sl = pl.ds(v * 16, 16)
                accv[sl] = accv[sl] + tmpv[sl]
                return c2

            lax.fori_loop(0, DEN_PER_SUB // 16, add_body, 0)
            return carry

        lax.fori_loop(0, 16, red_body, 0)
        pltpu.sync_copy(accv, den_h.at[pl.ds(ci * DEN_PAD + col0, DEN_PER_SUB)])

    return k(asrc2, adst2, amax32, srcp, dstp)


def _message_pass(h, srcp, dstp, ex0, ex1, zbig):
    mesh = plsc.VectorSubcoreMesh(core_axis_name="c", subcore_axis_name="s")

    @functools.partial(
        pl.kernel,
        out_type=jax.ShapeDtypeStruct((2 * N, HC), jnp.float32),
        mesh=mesh,
        compiler_params=pltpu.CompilerParams(needs_layout_passes=False),
        scratch_types=[pltpu.VMEM((CHUNK,), jnp.int32),
                       pltpu.VMEM((CHUNK,), jnp.int32),
                       pltpu.VMEM((CHUNK,), jnp.int32),
                       pltpu.VMEM((CHUNK,), jnp.int32),
                       pltpu.VMEM((CHUNK,), jnp.float32),
                       pltpu.VMEM((CHUNK,), jnp.float32),
                       pltpu.VMEM((CHUNK,), jnp.float32),
                       pltpu.VMEM((CHUNK,), jnp.float32),
                       pltpu.VMEM((CHUNK, HC), jnp.float32),
                       pltpu.VMEM((CHUNK, HC), jnp.float32),
                       pltpu.VMEM_SHARED((N, HC), jnp.float32),
                       pltpu.SemaphoreType.DMA, pltpu.SemaphoreType.DMA,
                       pltpu.SemaphoreType.DMA, pltpu.SemaphoreType.DMA],
    )
    def k(h_h, src_h, dst_h, ex0_h, ex1_h, z_h, out_h,
          srcbA, srcbB, dstbA, dstbB, ex0bA, ex1bA, ex0bB, ex1bB,
          rowsA, rowsB, acc_sp, rsemA, rsemB, gsemA, gsemB):
        ci = lax.axis_index("c")
        si = lax.axis_index("s")
        t = ci * 16 + si
        r0 = si * ROWS_PER_SUB
        TAIL0 = 16 * ROWS_PER_SUB
        TAILN = N - TAIL0
        pltpu.sync_copy(z_h.at[pl.ds(r0, ROWS_PER_SUB)],
                        acc_sp.at[pl.ds(r0, ROWS_PER_SUB)])

        @pl.when(si == 15)
        def _():
            pltpu.sync_copy(z_h.at[pl.ds(TAIL0, TAILN)], acc_sp.at[pl.ds(TAIL0, TAILN)])

        plsc.subcore_barrier()
        cbase = t * CPT

        def issue_idx(cg, srcb, dstb, ex0b, ex1b, rsem):
            base = cg * CHUNK
            pltpu.async_copy(src_h.at[pl.ds(base, CHUNK)], srcb, rsem)
            pltpu.async_copy(dst_h.at[pl.ds(base, CHUNK)], dstb, rsem)
            pltpu.async_copy(ex0_h.at[pl.ds(base, CHUNK)], ex0b, rsem)
            pltpu.async_copy(ex1_h.at[pl.ds(base, CHUNK)], ex1b, rsem)

        def wait_idx(srcb, dstb, ex0b, ex1b, rsem):
            pltpu.make_async_copy(src_h.at[pl.ds(0, CHUNK)], srcb, rsem).wait()
            pltpu.make_async_copy(dst_h.at[pl.ds(0, CHUNK)], dstb, rsem).wait()
            pltpu.make_async_copy(ex0_h.at[pl.ds(0, CHUNK)], ex0b, rsem).wait()
            pltpu.make_async_copy(ex1_h.at[pl.ds(0, CHUNK)], ex1b, rsem).wait()

        def scale(rows, ex0b, ex1b):
            def e_body(e, c2):
                idx = jnp.broadcast_to(e, (16,)).astype(jnp.int32)
                s0 = plsc.load_gather(ex0b, [idx])
                s1 = plsc.load_gather(ex1b, [idx])
                for j in range(HC // 16):
                    sc = s0 if j < (HC // 32) else s1
                    rows[e, pl.ds(j * 16, 16)] = rows[e, pl.ds(j * 16, 16)] * sc
                return c2

            lax.fori_loop(0, CHUNK, e_body, 0)

        issue_idx(cbase, srcbA, dstbA, ex0bA, ex1bA, rsemA)
        issue_idx(cbase + 1, srcbB, dstbB, ex0bB, ex1bB, rsemB)
        wait_idx(srcbA, dstbA, ex0bA, ex1bA, rsemA)
        pltpu.async_copy(h_h.at[srcbA], rowsA, gsemA)

        def pair_body(p, carry):
            c0 = cbase + 2 * p
            more = p < NPAIR - 1
            wait_idx(srcbB, dstbB, ex0bB, ex1bB, rsemB)
            pltpu.async_copy(h_h.at[srcbB], rowsB, gsemB)

            pltpu.make_async_copy(h_h.at[srcbA], rowsA, gsemA).wait()
            scale(rowsA, ex0bA, ex1bA)
            pltpu.sync_copy(rowsA, acc_sp.at[dstbA], add=True)

            @pl.when(more)
            def _():
                issue_idx(c0 + 2, srcbA, dstbA, ex0bA, ex1bA, rsemA)
                wait_idx(srcbA, dstbA, ex0bA, ex1bA, rsemA)
                pltpu.async_copy(h_h.at[srcbA], rowsA, gsemA)

            pltpu.make_async_copy(h_h.at[srcbB], rowsB, gsemB).wait()
            scale(rowsB, ex0bB, ex1bB)
            pltpu.sync_copy(rowsB, acc_sp.at[dstbB], add=True)

            @pl.when(more)
            def _():
                issue_idx(c0 + 3, srcbB, dstbB, ex0bB, ex1bB, rsemB)

            return carry

        lax.fori_loop(0, NPAIR, pair_body, 0)
        plsc.subcore_barrier()
        pltpu.sync_copy(acc_sp.at[pl.ds(r0, ROWS_PER_SUB)],
                        out_h.at[pl.ds(ci * N + r0, ROWS_PER_SUB)])

        @pl.when(si == 15)
        def _():
            pltpu.sync_copy(acc_sp.at[pl.ds(TAIL0, TAILN)],
                            out_h.at[pl.ds(ci * N + TAIL0, TAILN)])

    return k(h, srcp, dstp, ex0, ex1, zbig)


def _finalize_body(p0_ref, p1_ref, d0_ref, d1_ref, b_ref, o_ref):
    den = d0_ref[...] + d1_ref[...]
    p = p0_ref[...] + p1_ref[...]
    cols = lax.broadcasted_iota(jnp.int32, p.shape, 1)
    den0 = jnp.broadcast_to(den[:, 0:1], p.shape)
    den1 = jnp.broadcast_to(den[:, 1:2], p.shape)
    db = jnp.where(cols < OUT, den0, den1)
    o_ref[...] = p / (db + 1e-16) + b_ref[...]


def _finalize(P, denp, bias2):
    RB = 1000
    G = N // RB
    return pl.pallas_call(
        _finalize_body,
        grid=(G,),
        in_specs=[pl.BlockSpec((RB, HC), lambda i: (i, 0)),
                  pl.BlockSpec((RB, HC), lambda i: (i + G, 0)),
                  pl.BlockSpec((RB, 2), lambda i: (i, 0)),
                  pl.BlockSpec((RB, 2), lambda i: (i + G, 0)),
                  pl.BlockSpec((1, HC), lambda i: (0, 0))],
        out_specs=pl.BlockSpec((RB, HC), lambda i: (i, 0)),
        out_shape=jax.ShapeDtypeStruct((N, HC), jnp.float32),
    )(P, P, denp, denp, bias2)


def kernel(x, edge_index, W, att_src, att_dst, bias):
    h, asrc, adst, amax = _prologue(x, W, att_src.reshape(1, HC),
                                    att_dst.reshape(1, HC))
    loops = jnp.arange(N, dtype=jnp.int32)
    pad = jnp.zeros((EP - ET,), jnp.int32)
    srcp = jnp.concatenate([edge_index[0], loops, pad])
    dstp = jnp.concatenate([edge_index[1], loops, pad])
    ex0, ex1, denp = _edge_softmax(asrc.reshape(2 * N), adst.reshape(2 * N),
                                   amax.reshape(32), srcp, dstp)
    P = _message_pass(h, srcp, dstp, ex0, ex1, jnp.zeros((N, HC), jnp.float32))
    den2 = denp.reshape(2, DEN_PAD)[:, :2 * N].reshape(2 * N, 2)
    return _finalize(P, den2, bias.reshape(1, HC))


# speedup vs baseline: 89.8277x; 1.4631x over previous
import functools

import jax
import jax.numpy as jnp
from jax import lax
from jax.experimental import pallas as pl
from jax.experimental.pallas import tpu as pltpu
from jax.experimental.pallas import tpu_sc as plsc

N = 10000
E = 320000
IN_DIM = 128
HEADS = 2
OUT = 64
HC = HEADS * OUT

ET = E + N
TILES = 32
CHUNK = 128
CPT = 82
EP = TILES * CPT * CHUNK
NPAIR = CPT // 2
ROWS_PER_SUB = 624
DEN_PAD = 20480
DEN_PER_SUB = DEN_PAD // 16


def _prologue_body(x_ref, w_ref, as_ref, ad_ref, h_ref, asrc_ref, adst_ref, amax_ref):
    xb = x_ref[...]
    xf = jnp.where(xb > 0, xb, jnp.exp(xb) - 1.0)
    h = jnp.dot(xf, w_ref[...], preferred_element_type=jnp.float32)
    h_ref[...] = h
    ts = h * as_ref[...]
    td = h * ad_ref[...]
    a0 = jnp.sum(ts[:, :OUT], axis=1, keepdims=True)
    a1 = jnp.sum(ts[:, OUT:], axis=1, keepdims=True)
    d0 = jnp.sum(td[:, :OUT], axis=1, keepdims=True)
    d1 = jnp.sum(td[:, OUT:], axis=1, keepdims=True)
    asrc_ref[...] = jnp.concatenate([a0, a1], axis=1)
    adst_ref[...] = jnp.concatenate([d0, d1], axis=1)

    @pl.when(pl.program_id(0) == 0)
    def _():
        amax_ref[...] = jnp.full((2, 16), -1e30, jnp.float32)

    upd = jnp.concatenate(
        [jnp.full((1, 16), jnp.max(a0), jnp.float32),
         jnp.full((1, 16), jnp.max(a1), jnp.float32)], axis=0)
    amax_ref[...] = jnp.maximum(amax_ref[...], upd)


def _prologue(x, W, atts, attd):
    RB = 1000
    return pl.pallas_call(
        _prologue_body,
        grid=(N // RB,),
        in_specs=[pl.BlockSpec((RB, IN_DIM), lambda i: (i, 0)),
                  pl.BlockSpec((IN_DIM, HC), lambda i: (0, 0)),
                  pl.BlockSpec((1, HC), lambda i: (0, 0)),
                  pl.BlockSpec((1, HC), lambda i: (0, 0))],
        out_specs=[pl.BlockSpec((RB, HC), lambda i: (i, 0)),
                   pl.BlockSpec((RB, 2), lambda i: (i, 0)),
                   pl.BlockSpec((RB, 2), lambda i: (i, 0)),
                   pl.BlockSpec((2, 16), lambda i: (0, 0))],
        out_shape=[jax.ShapeDtypeStruct((N, HC), jnp.float32),
                   jax.ShapeDtypeStruct((N, 2), jnp.float32),
                   jax.ShapeDtypeStruct((N, 2), jnp.float32),
                   jax.ShapeDtypeStruct((2, 16), jnp.float32)],
    )(x, W, atts, attd)


def _edge_softmax(asrc2, adst2, amax32, srcp, dstp):
    mesh = plsc.VectorSubcoreMesh(core_axis_name="c", subcore_axis_name="s")

    @functools.partial(
        pl.kernel,
        out_type=[jax.ShapeDtypeStruct((EP,), jnp.float32),
                  jax.ShapeDtypeStruct((EP,), jnp.float32),
                  jax.ShapeDtypeStruct((2 * DEN_PAD,), jnp.float32)],
        mesh=mesh,
        compiler_params=pltpu.CompilerParams(needs_layout_passes=False),
        scratch_types=[pltpu.VMEM((2 * N,), jnp.float32),
                       pltpu.VMEM((2 * N,), jnp.float32),
                       pltpu.VMEM((32,), jnp.float32),
                       pltpu.VMEM((CHUNK,), jnp.int32),
                       pltpu.VMEM((CHUNK,), jnp.int32),
                       pltpu.VMEM((CHUNK,), jnp.float32),
                       pltpu.VMEM((CHUNK,), jnp.float32),
                       pltpu.VMEM((DEN_PAD,), jnp.float32),
                       pltpu.VMEM((DEN_PER_SUB,), jnp.float32),
                       pltpu.VMEM((DEN_PER_SUB,), jnp.float32),
                       pltpu.VMEM_SHARED((16 * DEN_PAD,), jnp.float32)],
    )
    def k(asrc_h, adst_h, amax_h, src_h, dst_h, ex0_h, ex1_h, den_h,
          asv, adv, amv, srcb, dstb, ex0b, ex1b, denv, accv, tmpv, den_sp):
        ci = lax.axis_index("c")
        si = lax.axis_index("s")
        t = ci * 16 + si
        pltpu.sync_copy(asrc_h, asv)
        pltpu.sync_copy(adst_h, adv)
        pltpu.sync_copy(amax_h, amv)

        z16f = jnp.zeros((16,), jnp.float32)

        def z_body(v, carry):
            denv[pl.ds(v * 16, 16)] = z16f
            return carry

        lax.fori_loop(0, DEN_PAD // 16, z_body, 0)

        am0 = amv[pl.ds(0, 16)]
        am1 = amv[pl.ds(16, 16)]
        iota = lax.iota(jnp.int32, 16)

        def chunk_body(c, carry):
            base = (t * CPT + c) * CHUNK
            pltpu.sync_copy(src_h.at[pl.ds(base, CHUNK)], srcb)
            pltpu.sync_copy(dst_h.at[pl.ds(base, CHUNK)], dstb)
            for g in range(CHUNK // 16):
                b16 = g * 16
                s16 = srcb[pl.ds(b16, 16)]
                d16 = dstb[pl.ds(b16, 16)]
                s2 = s16 * 2
                d2 = d16 * 2
                as0 = plsc.load_gather(asv, [s2])
                as1 = plsc.load_gather(asv, [s2 + 1])
                ad0 = plsc.load_gather(adv, [d2])
                ad1 = plsc.load_gather(adv, [d2 + 1])
                al0 = as0 + ad0
                al1 = as1 + ad1
                al0 = jnp.maximum(al0, 0.2 * al0)
                al1 = jnp.maximum(al1, 0.2 * al1)
                b0 = am0 + ad0
                b1 = am1 + ad1
                b0 = jnp.maximum(b0, 0.2 * b0)
                b1 = jnp.maximum(b1, 0.2 * b1)
                e0 = jnp.exp(al0 - b0)
                e1 = jnp.exp(al1 - b1)
                eid = base + b16 + iota
                msk = eid < ET
                e0 = jnp.where(msk, e0, 0.0)
                e1 = jnp.where(msk, e1, 0.0)
                ex0b[pl.ds(b16, 16)] = e0
                ex1b[pl.ds(b16, 16)] = e1
                plsc.addupdate_scatter(denv, [d2], e0)
                plsc.addupdate_scatter(denv, [d2 + 1], e1)
            pltpu.sync_copy(ex0b, ex0_h.at[pl.ds(base, CHUNK)])
            pltpu.sync_copy(ex1b, ex1_h.at[pl.ds(base, CHUNK)])
            return carry

        lax.fori_loop(0, CPT, chunk_body, 0)

        pltpu.sync_copy(denv, den_sp.at[pl.ds(si * DEN_PAD, DEN_PAD)])
        plsc.subcore_barrier()
        col0 = si * DEN_PER_SUB

        def za_body(v, carry):
            accv[pl.ds(v * 16, 16)] = z16f
            return carry

        lax.fori_loop(0, DEN_PER_SUB // 16, za_body, 0)

        def red_body(r, carry):
            pltpu.sync_copy(den_sp.at[pl.ds(r * DEN_PAD + col0, DEN_PER_SUB)], tmpv)

            def add_body(v, c2):
                sl = pl.ds(v * 16, 16)
                accv[sl] = accv[sl] + tmpv[sl]
                return c2

            lax.fori_loop(0, DEN_PER_SUB // 16, add_body, 0)
            return carry

        lax.fori_loop(0, 16, red_body, 0)
        pltpu.sync_copy(accv, den_h.at[pl.ds(ci * DEN_PAD + col0, DEN_PER_SUB)])

    return k(asrc2, adst2, amax32, srcp, dstp)


def _message_pass(h, srcp, dstp, ex0, ex1, zbig):
    mesh = plsc.VectorSubcoreMesh(core_axis_name="c", subcore_axis_name="s")

    @functools.partial(
        pl.kernel,
        out_type=jax.ShapeDtypeStruct((2 * N, HC), jnp.float32),
        mesh=mesh,
        compiler_params=pltpu.CompilerParams(needs_layout_passes=False),
        scratch_types=[pltpu.VMEM((CHUNK,), jnp.int32),
                       pltpu.VMEM((CHUNK,), jnp.int32),
                       pltpu.VMEM((CHUNK,), jnp.int32),
                       pltpu.VMEM((CHUNK,), jnp.int32),
                       pltpu.VMEM((CHUNK,), jnp.float32),
                       pltpu.VMEM((CHUNK,), jnp.float32),
                       pltpu.VMEM((CHUNK,), jnp.float32),
                       pltpu.VMEM((CHUNK,), jnp.float32),
                       pltpu.VMEM((CHUNK, HC), jnp.float32),
                       pltpu.VMEM((CHUNK, HC), jnp.float32),
                       pltpu.VMEM_SHARED((N, HC), jnp.float32),
                       pltpu.SemaphoreType.DMA, pltpu.SemaphoreType.DMA,
                       pltpu.SemaphoreType.DMA, pltpu.SemaphoreType.DMA],
    )
    def k(h_h, src_h, dst_h, ex0_h, ex1_h, z_h, out_h,
          srcbA, srcbB, dstbA, dstbB, ex0bA, ex1bA, ex0bB, ex1bB,
          rowsA, rowsB, acc_sp, rsemA, rsemB, gsemA, gsemB):
        ci = lax.axis_index("c")
        si = lax.axis_index("s")
        t = ci * 16 + si
        r0 = si * ROWS_PER_SUB
        TAIL0 = 16 * ROWS_PER_SUB
        TAILN = N - TAIL0
        pltpu.sync_copy(z_h.at[pl.ds(r0, ROWS_PER_SUB)],
                        acc_sp.at[pl.ds(r0, ROWS_PER_SUB)])

        @pl.when(si == 15)
        def _():
            pltpu.sync_copy(z_h.at[pl.ds(TAIL0, TAILN)], acc_sp.at[pl.ds(TAIL0, TAILN)])

        plsc.subcore_barrier()
        cbase = t * CPT

        def issue_idx(cg, srcb, dstb, ex0b, ex1b, rsem):
            base = cg * CHUNK
            pltpu.async_copy(src_h.at[pl.ds(base, CHUNK)], srcb, rsem)
            pltpu.async_copy(dst_h.at[pl.ds(base, CHUNK)], dstb, rsem)
            pltpu.async_copy(ex0_h.at[pl.ds(base, CHUNK)], ex0b, rsem)
            pltpu.async_copy(ex1_h.at[pl.ds(base, CHUNK)], ex1b, rsem)

        def wait_idx(srcb, dstb, ex0b, ex1b, rsem):
            pltpu.make_async_copy(src_h.at[pl.ds(0, CHUNK)], srcb, rsem).wait()
            pltpu.make_async_copy(dst_h.at[pl.ds(0, CHUNK)], dstb, rsem).wait()
            pltpu.make_async_copy(ex0_h.at[pl.ds(0, CHUNK)], ex0b, rsem).wait()
            pltpu.make_async_copy(ex1_h.at[pl.ds(0, CHUNK)], ex1b, rsem).wait()

        def scale(rows, ex0b, ex1b):
            def e_body(e, c2):
                idx = jnp.broadcast_to(e, (16,)).astype(jnp.int32)
                s0 = plsc.load_gather(ex0b, [idx])
                s1 = plsc.load_gather(ex1b, [idx])
                for j in range(HC // 16):
                    sc = s0 if j < (HC // 32) else s1
                    rows[e, pl.ds(j * 16, 16)] = rows[e, pl.ds(j * 16, 16)] * sc
                return c2

            lax.fori_loop(0, CHUNK, e_body, 0)

        issue_idx(cbase, srcbA, dstbA, ex0bA, ex1bA, rsemA)
        issue_idx(cbase + 1, srcbB, dstbB, ex0bB, ex1bB, rsemB)
        wait_idx(srcbA, dstbA, ex0bA, ex1bA, rsemA)
        pltpu.async_copy(h_h.at[srcbA], rowsA, gsemA)

        def live(cg):
            return cg * CHUNK < ET

        def pair_body(p, carry):
            c0 = cbase + 2 * p
            more = p < NPAIR - 1
            wait_idx(srcbB, dstbB, ex0bB, ex1bB, rsemB)

            @pl.when(live(c0 + 1))
            def _():
                pltpu.async_copy(h_h.at[srcbB], rowsB, gsemB)

            @pl.when(live(c0))
            def _():
                pltpu.make_async_copy(h_h.at[srcbA], rowsA, gsemA).wait()
                scale(rowsA, ex0bA, ex1bA)
                pltpu.sync_copy(rowsA, acc_sp.at[dstbA], add=True)

            @pl.when(more)
            def _():
                issue_idx(c0 + 2, srcbA, dstbA, ex0bA, ex1bA, rsemA)
                wait_idx(srcbA, dstbA, ex0bA, ex1bA, rsemA)

                @pl.when(live(c0 + 2))
                def _():
                    pltpu.async_copy(h_h.at[srcbA], rowsA, gsemA)

            @pl.when(live(c0 + 1))
            def _():
                pltpu.make_async_copy(h_h.at[srcbB], rowsB, gsemB).wait()
                scale(rowsB, ex0bB, ex1bB)
                pltpu.sync_copy(rowsB, acc_sp.at[dstbB], add=True)

            @pl.when(more)
            def _():
                issue_idx(c0 + 3, srcbB, dstbB, ex0bB, ex1bB, rsemB)

            return carry

        lax.fori_loop(0, NPAIR, pair_body, 0)
        plsc.subcore_barrier()
        pltpu.sync_copy(acc_sp.at[pl.ds(r0, ROWS_PER_SUB)],
                        out_h.at[pl.ds(ci * N + r0, ROWS_PER_SUB)])

        @pl.when(si == 15)
        def _():
            pltpu.sync_copy(acc_sp.at[pl.ds(TAIL0, TAILN)],
                            out_h.at[pl.ds(ci * N + TAIL0, TAILN)])

    return k(h, srcp, dstp, ex0, ex1, zbig)


def _finalize_body(p0_ref, p1_ref, d0_ref, d1_ref, b_ref, o_ref):
    den = d0_ref[...] + d1_ref[...]
    p = p0_ref[...] + p1_ref[...]
    cols = lax.broadcasted_iota(jnp.int32, p.shape, 1)
    den0 = jnp.broadcast_to(den[:, 0:1], p.shape)
    den1 = jnp.broadcast_to(den[:, 1:2], p.shape)
    db = jnp.where(cols < OUT, den0, den1)
    o_ref[...] = p / (db + 1e-16) + b_ref[...]


def _finalize(P, denp, bias2):
    RB = 1000
    G = N // RB
    return pl.pallas_call(
        _finalize_body,
        grid=(G,),
        in_specs=[pl.BlockSpec((RB, HC), lambda i: (i, 0)),
                  pl.BlockSpec((RB, HC), lambda i: (i + G, 0)),
                  pl.BlockSpec((RB, 2), lambda i: (i, 0)),
                  pl.BlockSpec((RB, 2), lambda i: (i + G, 0)),
                  pl.BlockSpec((1, HC), lambda i: (0, 0))],
        out_specs=pl.BlockSpec((RB, HC), lambda i: (i, 0)),
        out_shape=jax.ShapeDtypeStruct((N, HC), jnp.float32),
    )(P, P, denp, denp, bias2)


def kernel(x, edge_index, W, att_src, att_dst, bias):
    h, asrc, adst, amax = _prologue(x, W, att_src.reshape(1, HC),
                                    att_dst.reshape(1, HC))
    loops = jnp.arange(N, dtype=jnp.int32)
    pad = jnp.zeros((EP - ET,), jnp.int32)
    srcp = jnp.concatenate([edge_index[0], loops, pad])
    dstp = jnp.concatenate([edge_index[1], loops, pad])
    ex0, ex1, denp = _edge_softmax(asrc.reshape(2 * N), adst.reshape(2 * N),
                                   amax.reshape(32), srcp, dstp)
    P = _message_pass(h, srcp, dstp, ex0, ex1, jnp.zeros((N, HC), jnp.float32))
    den2 = denp.reshape(2, DEN_PAD)[:, :2 * N].reshape(2 * N, 2)
    return _finalize(P, den2, bias.reshape(1, HC))


# speedup vs baseline: 105.4749x; 1.1742x over previous
import functools

import jax
import jax.numpy as jnp
from jax import lax
from jax.experimental import pallas as pl
from jax.experimental.pallas import tpu as pltpu
from jax.experimental.pallas import tpu_sc as plsc

N = 10000
E = 320000
IN_DIM = 128
HEADS = 2
OUT = 64
HC = HEADS * OUT

ET = E + N
TILES = 32
CHUNK = 128
CPT = 82
EP = TILES * CPT * CHUNK
NPAIR = CPT // 2
ROWS_PER_SUB = 624
DEN_PAD = 20480
DEN_PER_SUB = DEN_PAD // 16


def _prologue_body(x_ref, w_ref, as_ref, ad_ref, h_ref, asrc_ref, adst_ref, amax_ref):
    xb = x_ref[...]
    xf = jnp.where(xb > 0, xb, jnp.exp(xb) - 1.0)
    h = jnp.dot(xf, w_ref[...], preferred_element_type=jnp.float32)
    h_ref[...] = h
    ts = h * as_ref[...]
    td = h * ad_ref[...]
    a0 = jnp.sum(ts[:, :OUT], axis=1, keepdims=True)
    a1 = jnp.sum(ts[:, OUT:], axis=1, keepdims=True)
    d0 = jnp.sum(td[:, :OUT], axis=1, keepdims=True)
    d1 = jnp.sum(td[:, OUT:], axis=1, keepdims=True)
    asrc_ref[...] = jnp.concatenate([a0, a1], axis=1)
    adst_ref[...] = jnp.concatenate([d0, d1], axis=1)

    @pl.when(pl.program_id(0) == 0)
    def _():
        amax_ref[...] = jnp.full((2, 16), -1e30, jnp.float32)

    upd = jnp.concatenate(
        [jnp.full((1, 16), jnp.max(a0), jnp.float32),
         jnp.full((1, 16), jnp.max(a1), jnp.float32)], axis=0)
    amax_ref[...] = jnp.maximum(amax_ref[...], upd)


def _prologue(x, W, atts, attd):
    RB = 1000
    return pl.pallas_call(
        _prologue_body,
        grid=(N // RB,),
        in_specs=[pl.BlockSpec((RB, IN_DIM), lambda i: (i, 0)),
                  pl.BlockSpec((IN_DIM, HC), lambda i: (0, 0)),
                  pl.BlockSpec((1, HC), lambda i: (0, 0)),
                  pl.BlockSpec((1, HC), lambda i: (0, 0))],
        out_specs=[pl.BlockSpec((RB, HC), lambda i: (i, 0)),
                   pl.BlockSpec((RB, 2), lambda i: (i, 0)),
                   pl.BlockSpec((RB, 2), lambda i: (i, 0)),
                   pl.BlockSpec((2, 16), lambda i: (0, 0))],
        out_shape=[jax.ShapeDtypeStruct((N, HC), jnp.float32),
                   jax.ShapeDtypeStruct((N, 2), jnp.float32),
                   jax.ShapeDtypeStruct((N, 2), jnp.float32),
                   jax.ShapeDtypeStruct((2, 16), jnp.float32)],
    )(x, W, atts, attd)


def _edge_softmax(asrc2, adst2, amax32, srcp, dstp):
    mesh = plsc.VectorSubcoreMesh(core_axis_name="c", subcore_axis_name="s")

    @functools.partial(
        pl.kernel,
        out_type=[jax.ShapeDtypeStruct((EP,), jnp.float32),
                  jax.ShapeDtypeStruct((EP,), jnp.float32),
                  jax.ShapeDtypeStruct((2 * DEN_PAD,), jnp.float32)],
        mesh=mesh,
        compiler_params=pltpu.CompilerParams(needs_layout_passes=False),
        scratch_types=[pltpu.VMEM((2 * N,), jnp.float32),
                       pltpu.VMEM((2 * N,), jnp.float32),
                       pltpu.VMEM((32,), jnp.float32),
                       pltpu.VMEM((CHUNK,), jnp.int32),
                       pltpu.VMEM((CHUNK,), jnp.int32),
                       pltpu.VMEM((CHUNK,), jnp.int32),
                       pltpu.VMEM((CHUNK,), jnp.int32),
                       pltpu.VMEM((CHUNK,), jnp.float32),
                       pltpu.VMEM((CHUNK,), jnp.float32),
                       pltpu.VMEM((DEN_PAD,), jnp.float32),
                       pltpu.VMEM((DEN_PER_SUB,), jnp.float32),
                       pltpu.VMEM((DEN_PER_SUB,), jnp.float32),
                       pltpu.VMEM_SHARED((16 * DEN_PAD,), jnp.float32),
                       pltpu.SemaphoreType.DMA, pltpu.SemaphoreType.DMA],
    )
    def k(asrc_h, adst_h, amax_h, src_h, dst_h, ex0_h, ex1_h, den_h,
          asv, adv, amv, srcbA, srcbB, dstbA, dstbB, ex0b, ex1b,
          denv, accv, tmpv, den_sp, rsemA, rsemB):
        ci = lax.axis_index("c")
        si = lax.axis_index("s")
        t = ci * 16 + si
        pltpu.sync_copy(asrc_h, asv)
        pltpu.sync_copy(adst_h, adv)
        pltpu.sync_copy(amax_h, amv)

        z16f = jnp.zeros((16,), jnp.float32)

        def z_body(v, carry):
            denv[pl.ds(v * 16, 16)] = z16f
            return carry

        lax.fori_loop(0, DEN_PAD // 16, z_body, 0)

        am0 = amv[pl.ds(0, 16)]
        am1 = amv[pl.ds(16, 16)]
        iota = lax.iota(jnp.int32, 16)
        cbase = t * CPT

        def issue_reads(cg, srcb, dstb, rsem):
            base = cg * CHUNK
            pltpu.async_copy(src_h.at[pl.ds(base, CHUNK)], srcb, rsem)
            pltpu.async_copy(dst_h.at[pl.ds(base, CHUNK)], dstb, rsem)

        def wait_reads(srcb, dstb, rsem):
            pltpu.make_async_copy(src_h.at[pl.ds(0, CHUNK)], srcb, rsem).wait()
            pltpu.make_async_copy(dst_h.at[pl.ds(0, CHUNK)], dstb, rsem).wait()

        def chunk_work(cg, srcb, dstb):
            base = cg * CHUNK
            for g in range(CHUNK // 16):
                b16 = g * 16
                s16 = srcb[pl.ds(b16, 16)]
                d16 = dstb[pl.ds(b16, 16)]
                s2 = s16 * 2
                d2 = d16 * 2
                as0 = plsc.load_gather(asv, [s2])
                as1 = plsc.load_gather(asv, [s2 + 1])
                ad0 = plsc.load_gather(adv, [d2])
                ad1 = plsc.load_gather(adv, [d2 + 1])
                al0 = as0 + ad0
                al1 = as1 + ad1
                al0 = jnp.maximum(al0, 0.2 * al0)
                al1 = jnp.maximum(al1, 0.2 * al1)
                b0 = am0 + ad0
                b1 = am1 + ad1
                b0 = jnp.maximum(b0, 0.2 * b0)
                b1 = jnp.maximum(b1, 0.2 * b1)
                e0 = jnp.exp(al0 - b0)
                e1 = jnp.exp(al1 - b1)
                eid = base + b16 + iota
                msk = eid < ET
                e0 = jnp.where(msk, e0, 0.0)
                e1 = jnp.where(msk, e1, 0.0)
                ex0b[pl.ds(b16, 16)] = e0
                ex1b[pl.ds(b16, 16)] = e1
                plsc.addupdate_scatter(denv, [d2], e0)
                plsc.addupdate_scatter(denv, [d2 + 1], e1)
            pltpu.sync_copy(ex0b, ex0_h.at[pl.ds(base, CHUNK)])
            pltpu.sync_copy(ex1b, ex1_h.at[pl.ds(base, CHUNK)])

        issue_reads(cbase, srcbA, dstbA, rsemA)
        issue_reads(cbase + 1, srcbB, dstbB, rsemB)

        def bpair_body(p, carry):
            c0 = cbase + 2 * p
            more = p < NPAIR - 1
            wait_reads(srcbA, dstbA, rsemA)
            chunk_work(c0, srcbA, dstbA)

            @pl.when(more)
            def _():
                issue_reads(c0 + 2, srcbA, dstbA, rsemA)

            wait_reads(srcbB, dstbB, rsemB)
            chunk_work(c0 + 1, srcbB, dstbB)

            @pl.when(more)
            def _():
                issue_reads(c0 + 3, srcbB, dstbB, rsemB)

            return carry

        lax.fori_loop(0, NPAIR, bpair_body, 0)

        pltpu.sync_copy(denv, den_sp.at[pl.ds(si * DEN_PAD, DEN_PAD)])
        plsc.subcore_barrier()
        col0 = si * DEN_PER_SUB

        def za_body(v, carry):
            accv[pl.ds(v * 16, 16)] = z16f
            return carry

        lax.fori_loop(0, DEN_PER_SUB // 16, za_body, 0)

        def red_body(r, carry):
            pltpu.sync_copy(den_sp.at[pl.ds(r * DEN_PAD + col0, DEN_PER_SUB)], tmpv)

            def add_body(v, c2):
                sl = pl.ds(v * 16, 16)
                accv[sl] = accv[sl] + tmpv[sl]
                return c2

            lax.fori_loop(0, DEN_PER_SUB // 16, add_body, 0)
            return carry

        lax.fori_loop(0, 16, red_body, 0)
        pltpu.sync_copy(accv, den_h.at[pl.ds(ci * DEN_PAD + col0, DEN_PER_SUB)])

    return k(asrc2, adst2, amax32, srcp, dstp)


def _message_pass(h, srcp, dstp, ex0, ex1, zbig):
    mesh = plsc.VectorSubcoreMesh(core_axis_name="c", subcore_axis_name="s")

    @functools.partial(
        pl.kernel,
        out_type=jax.ShapeDtypeStruct((2 * N, HC), jnp.float32),
        mesh=mesh,
        compiler_params=pltpu.CompilerParams(needs_layout_passes=False),
        scratch_types=[pltpu.VMEM((CHUNK,), jnp.int32),
                       pltpu.VMEM((CHUNK,), jnp.int32),
                       pltpu.VMEM((CHUNK,), jnp.int32),
                       pltpu.VMEM((CHUNK,), jnp.int32),
                       pltpu.VMEM((CHUNK,), jnp.float32),
                       pltpu.VMEM((CHUNK,), jnp.float32),
                       pltpu.VMEM((CHUNK,), jnp.float32),
                       pltpu.VMEM((CHUNK,), jnp.float32),
                       pltpu.VMEM((CHUNK, HC), jnp.float32),
                       pltpu.VMEM((CHUNK, HC), jnp.float32),
                       pltpu.VMEM_SHARED((N, HC), jnp.float32),
                       pltpu.SemaphoreType.DMA, pltpu.SemaphoreType.DMA,
                       pltpu.SemaphoreType.DMA, pltpu.SemaphoreType.DMA],
    )
    def k(h_h, src_h, dst_h, ex0_h, ex1_h, z_h, out_h,
          srcbA, srcbB, dstbA, dstbB, ex0bA, ex1bA, ex0bB, ex1bB,
          rowsA, rowsB, acc_sp, rsemA, rsemB, gsemA, gsemB):
        ci = lax.axis_index("c")
        si = lax.axis_index("s")
        t = ci * 16 + si
        r0 = si * ROWS_PER_SUB
        TAIL0 = 16 * ROWS_PER_SUB
        TAILN = N - TAIL0
        pltpu.sync_copy(z_h.at[pl.ds(r0, ROWS_PER_SUB)],
                        acc_sp.at[pl.ds(r0, ROWS_PER_SUB)])

        @pl.when(si == 15)
        def _():
            pltpu.sync_copy(z_h.at[pl.ds(TAIL0, TAILN)], acc_sp.at[pl.ds(TAIL0, TAILN)])

        plsc.subcore_barrier()
        cbase = t * CPT

        def issue_idx(cg, srcb, dstb, ex0b, ex1b, rsem):
            base = cg * CHUNK
            pltpu.async_copy(src_h.at[pl.ds(base, CHUNK)], srcb, rsem)
            pltpu.async_copy(dst_h.at[pl.ds(base, CHUNK)], dstb, rsem)
            pltpu.async_copy(ex0_h.at[pl.ds(base, CHUNK)], ex0b, rsem)
            pltpu.async_copy(ex1_h.at[pl.ds(base, CHUNK)], ex1b, rsem)

        def wait_idx(srcb, dstb, ex0b, ex1b, rsem):
            pltpu.make_async_copy(src_h.at[pl.ds(0, CHUNK)], srcb, rsem).wait()
            pltpu.make_async_copy(dst_h.at[pl.ds(0, CHUNK)], dstb, rsem).wait()
            pltpu.make_async_copy(ex0_h.at[pl.ds(0, CHUNK)], ex0b, rsem).wait()
            pltpu.make_async_copy(ex1_h.at[pl.ds(0, CHUNK)], ex1b, rsem).wait()

        def scale(rows, ex0b, ex1b):
            def e_body(e, c2):
                idx = jnp.broadcast_to(e, (16,)).astype(jnp.int32)
                s0 = plsc.load_gather(ex0b, [idx])
                s1 = plsc.load_gather(ex1b, [idx])
                for j in range(HC // 16):
                    sc = s0 if j < (HC // 32) else s1
                    rows[e, pl.ds(j * 16, 16)] = rows[e, pl.ds(j * 16, 16)] * sc
                return c2

            lax.fori_loop(0, CHUNK, e_body, 0)

        issue_idx(cbase, srcbA, dstbA, ex0bA, ex1bA, rsemA)
        issue_idx(cbase + 1, srcbB, dstbB, ex0bB, ex1bB, rsemB)
        wait_idx(srcbA, dstbA, ex0bA, ex1bA, rsemA)
        pltpu.async_copy(h_h.at[srcbA], rowsA, gsemA)

        def live(cg):
            return cg * CHUNK < ET

        def pair_body(p, carry):
            c0 = cbase + 2 * p
            more = p < NPAIR - 1
            wait_idx(srcbB, dstbB, ex0bB, ex1bB, rsemB)

            @pl.when(live(c0 + 1))
            def _():
                pltpu.async_copy(h_h.at[srcbB], rowsB, gsemB)

            @pl.when(live(c0))
            def _():
                pltpu.make_async_copy(h_h.at[srcbA], rowsA, gsemA).wait()
                scale(rowsA, ex0bA, ex1bA)
                pltpu.sync_copy(rowsA, acc_sp.at[dstbA], add=True)

            @pl.when(more)
            def _():
                issue_idx(c0 + 2, srcbA, dstbA, ex0bA, ex1bA, rsemA)
                wait_idx(srcbA, dstbA, ex0bA, ex1bA, rsemA)

                @pl.when(live(c0 + 2))
                def _():
                    pltpu.async_copy(h_h.at[srcbA], rowsA, gsemA)

            @pl.when(live(c0 + 1))
            def _():
                pltpu.make_async_copy(h_h.at[srcbB], rowsB, gsemB).wait()
                scale(rowsB, ex0bB, ex1bB)
                pltpu.sync_copy(rowsB, acc_sp.at[dstbB], add=True)

            @pl.when(more)
            def _():
                issue_idx(c0 + 3, srcbB, dstbB, ex0bB, ex1bB, rsemB)

            return carry

        lax.fori_loop(0, NPAIR, pair_body, 0)
        plsc.subcore_barrier()
        pltpu.sync_copy(acc_sp.at[pl.ds(r0, ROWS_PER_SUB)],
                        out_h.at[pl.ds(ci * N + r0, ROWS_PER_SUB)])

        @pl.when(si == 15)
        def _():
            pltpu.sync_copy(acc_sp.at[pl.ds(TAIL0, TAILN)],
                            out_h.at[pl.ds(ci * N + TAIL0, TAILN)])

    return k(h, srcp, dstp, ex0, ex1, zbig)


def _finalize_body(p0_ref, p1_ref, d0_ref, d1_ref, b_ref, o_ref):
    den = d0_ref[...] + d1_ref[...]
    p = p0_ref[...] + p1_ref[...]
    cols = lax.broadcasted_iota(jnp.int32, p.shape, 1)
    den0 = jnp.broadcast_to(den[:, 0:1], p.shape)
    den1 = jnp.broadcast_to(den[:, 1:2], p.shape)
    db = jnp.where(cols < OUT, den0, den1)
    o_ref[...] = p / (db + 1e-16) + b_ref[...]


def _finalize(P, denp, bias2):
    RB = 1000
    G = N // RB
    return pl.pallas_call(
        _finalize_body,
        grid=(G,),
        in_specs=[pl.BlockSpec((RB, HC), lambda i: (i, 0)),
                  pl.BlockSpec((RB, HC), lambda i: (i + G, 0)),
                  pl.BlockSpec((RB, 2), lambda i: (i, 0)),
                  pl.BlockSpec((RB, 2), lambda i: (i + G, 0)),
                  pl.BlockSpec((1, HC), lambda i: (0, 0))],
        out_specs=pl.BlockSpec((RB, HC), lambda i: (i, 0)),
        out_shape=jax.ShapeDtypeStruct((N, HC), jnp.float32),
    )(P, P, denp, denp, bias2)


def kernel(x, edge_index, W, att_src, att_dst, bias):
    h, asrc, adst, amax = _prologue(x, W, att_src.reshape(1, HC),
                                    att_dst.reshape(1, HC))
    loops = jnp.arange(N, dtype=jnp.int32)
    pad = jnp.zeros((EP - ET,), jnp.int32)
    srcp = jnp.concatenate([edge_index[0], loops, pad])
    dstp = jnp.concatenate([edge_index[1], loops, pad])
    ex0, ex1, denp = _edge_softmax(asrc.reshape(2 * N), adst.reshape(2 * N),
                                   amax.reshape(32), srcp, dstp)
    P = _message_pass(h, srcp, dstp, ex0, ex1, jnp.zeros((N, HC), jnp.float32))
    den2 = denp.reshape(2, DEN_PAD)[:, :2 * N].reshape(2 * N, 2)
    return _finalize(P, den2, bias.reshape(1, HC))
